# Initial kernel scaffold; baseline (speedup 1.0000x reference)
import jax
import jax.numpy as jnp
from jax.experimental import pallas as pl


def kernel(positions, scales, rotations, distances, pe, scale_table, rotation_table, distance_table, fusion_weights):
    raise NotImplementedError("write your pallas kernel here")


# speedup vs baseline: 1.2818x; 1.2818x over previous
import functools
import math

import jax
import jax.numpy as jnp
from jax import lax
from jax.experimental import pallas as pl
from jax.experimental.pallas import tpu as pltpu
from jax.experimental.pallas import tpu_sc as plsc

NC = 2
NS = 16
L = 16


@functools.partial(jax.jit, static_argnames=("n_rows", "d", "seq_len"))
def _sc_fused_lookup(idx_s, idx_r, idx_d, pe, st, rt, dt, wvec, *, n_rows, d, seq_len):
    NW = NC * NS
    C = 16
    rows_per_w = n_rows // NW
    n_chunks = rows_per_w // C
    groups = d // L

    mesh = plsc.VectorSubcoreMesh(
        core_axis_name="c", subcore_axis_name="s",
        num_cores=NC, num_subcores=NS)

    @functools.partial(
        pl.kernel,
        out_type=jax.ShapeDtypeStruct((n_rows, d), jnp.float32),
        mesh=mesh,
        scratch_types=[
            pltpu.VMEM((n_chunks, C), jnp.int32),
            pltpu.VMEM((n_chunks, C), jnp.int32),
            pltpu.VMEM((n_chunks, C), jnp.int32),
            pltpu.VMEM((C, d), jnp.float32),
            pltpu.VMEM((C, d), jnp.float32),
            pltpu.VMEM((C, d), jnp.float32),
            pltpu.VMEM((C, d), jnp.float32),
            pltpu.VMEM((C, d), jnp.float32),
            pltpu.VMEM((4, L), jnp.float32),
            pltpu.SemaphoreType.DMA,
        ],
    )
    def body(sc_hbm, ro_hbm, di_hbm, pe_hbm, st_hbm, rt_hbm, dt_hbm, w_hbm,
             out_hbm, idx_sv, idx_rv, idx_dv, pe_v, g1, g2, g3, acc, w_v, sem):
        wid = lax.axis_index("s") * NC + lax.axis_index("c")
        base = wid * rows_per_w
        s_base = lax.rem(base, seq_len)

        cbase = wid * n_chunks
        pltpu.sync_copy(sc_hbm.at[pl.ds(cbase, n_chunks)], idx_sv)
        pltpu.sync_copy(ro_hbm.at[pl.ds(cbase, n_chunks)], idx_rv)
        pltpu.sync_copy(di_hbm.at[pl.ds(cbase, n_chunks)], idx_dv)
        pltpu.sync_copy(w_hbm, w_v)
        w0 = w_v[0, :]
        w1 = w_v[1, :]
        w2 = w_v[2, :]
        w3 = w_v[3, :]

        def chunk(t, carry):
            row0 = base + t * C
            s0 = s_base + t * C
            cp1 = pltpu.async_copy(st_hbm.at[idx_sv.at[t]], g1, sem)
            cp2 = pltpu.async_copy(rt_hbm.at[idx_rv.at[t]], g2, sem)
            cp3 = pltpu.async_copy(dt_hbm.at[idx_dv.at[t]], g3, sem)
            cp4 = pltpu.async_copy(pe_hbm.at[pl.ds(s0, C)], pe_v, sem)
            cp1.wait()
            cp2.wait()
            cp3.wait()
            cp4.wait()

            def row(i, carry2):
                def grp(j, carry3):
                    sl = pl.ds(j * L, L)
                    acc[i, sl] = (pe_v[i, sl] * w0 + g1[i, sl] * w1
                                  + g2[i, sl] * w2 + g3[i, sl] * w3)
                    return carry3
                return lax.fori_loop(0, groups, grp, carry2)

            lax.fori_loop(0, C, row, 0)
            pltpu.sync_copy(acc, out_hbm.at[pl.ds(row0, C)])
            return carry

        lax.fori_loop(0, n_chunks, chunk, 0)

    return body(idx_s, idx_r, idx_d, pe, st, rt, dt, wvec)


def kernel(positions, scales, rotations, distances, pe, scale_table,
           rotation_table, distance_table, fusion_weights):
    b, s = positions.shape
    d = pe.shape[1]
    n = b * s
    C = 16
    w = jax.nn.softmax(fusion_weights.astype(jnp.float32), axis=0)
    wvec = jnp.broadcast_to(w[:, None], (4, L)).astype(jnp.float32)
    idx_s = scales.reshape(n // C, C).astype(jnp.int32)
    idx_r = rotations.reshape(n // C, C).astype(jnp.int32)
    idx_d = distances.reshape(n // C, C).astype(jnp.int32)
    out = _sc_fused_lookup(idx_s, idx_r, idx_d, pe, scale_table,
                           rotation_table, distance_table, wvec,
                           n_rows=n, d=d, seq_len=s)
    return out.reshape(b, s, d)


# speedup vs baseline: 2.3549x; 1.8372x over previous
import functools

import jax
import jax.numpy as jnp
from jax import lax
from jax.experimental import pallas as pl
from jax.experimental.pallas import tpu as pltpu
from jax.experimental.pallas import tpu_sc as plsc

NC = 2
NS = 16
L = 16
C = 16


@functools.partial(jax.jit, static_argnames=("batch", "seq_len", "d"))
def _sc_fused_lookup(idx_s, idx_r, idx_d, pe, st, rt, dt, wvec, *, batch, seq_len, d):
    NW = NC * NS
    s_per_w = seq_len // NW
    n_sc = s_per_w // C
    n_stage = n_sc * batch
    groups = d // L

    mesh = plsc.VectorSubcoreMesh(
        core_axis_name="c", subcore_axis_name="s",
        num_cores=NC, num_subcores=NS)

    @functools.partial(
        pl.kernel,
        out_type=jax.ShapeDtypeStruct((batch * seq_len, d), jnp.float32),
        mesh=mesh,
        scratch_types=[
            pltpu.VMEM((batch, n_sc, C), jnp.int32),
            pltpu.VMEM((batch, n_sc, C), jnp.int32),
            pltpu.VMEM((batch, n_sc, C), jnp.int32),
            pltpu.VMEM((C, d), jnp.float32),
            pltpu.VMEM((2, C, d), jnp.float32),
            pltpu.VMEM((2, C, d), jnp.float32),
            pltpu.VMEM((2, C, d), jnp.float32),
            pltpu.VMEM((4, L), jnp.float32),
            pltpu.SemaphoreType.DMA,
            pltpu.SemaphoreType.DMA,
            pltpu.SemaphoreType.DMA,
            pltpu.SemaphoreType.DMA,
            pltpu.SemaphoreType.DMA,
        ],
    )
    def body(sc_hbm, ro_hbm, di_hbm, pe_hbm, st_hbm, rt_hbm, dt_hbm, w_hbm,
             out_hbm, idx_sv, idx_rv, idx_dv, pe_v, g1, g2, g3, w_v,
             sem_g0, sem_g1, sem_o0, sem_o1, sem_pe):
        wid = lax.axis_index("s") * NC + lax.axis_index("c")
        s_base = wid * s_per_w

        pltpu.sync_copy(sc_hbm.at[wid], idx_sv)
        pltpu.sync_copy(ro_hbm.at[wid], idx_rv)
        pltpu.sync_copy(di_hbm.at[wid], idx_dv)
        pltpu.sync_copy(w_hbm, w_v)
        w0 = w_v[0, :]
        w1 = w_v[1, :]
        w2 = w_v[2, :]
        w3 = w_v[3, :]

        sem_g = (sem_g0, sem_g1)
        sem_o = (sem_o0, sem_o1)
        gbufs = ((g1.at[0], g2.at[0], g3.at[0]), (g1.at[1], g2.at[1], g3.at[1]))

        def stage_tb(ls):
            return ls // batch, lax.rem(ls, batch)

        def issue_gathers(ls, p):
            t, b = stage_tb(ls)
            b1, b2, b3 = gbufs[p]
            pltpu.async_copy(st_hbm.at[idx_sv.at[b, t]], b1, sem_g[p])
            pltpu.async_copy(rt_hbm.at[idx_rv.at[b, t]], b2, sem_g[p])
            pltpu.async_copy(dt_hbm.at[idx_dv.at[b, t]], b3, sem_g[p])

        def wait_gathers(ls, p):
            t, b = stage_tb(ls)
            b1, b2, b3 = gbufs[p]
            pltpu.make_async_copy(st_hbm.at[idx_sv.at[b, t]], b1, sem_g[p]).wait()
            pltpu.make_async_copy(rt_hbm.at[idx_rv.at[b, t]], b2, sem_g[p]).wait()
            pltpu.make_async_copy(dt_hbm.at[idx_dv.at[b, t]], b3, sem_g[p]).wait()

        def out_rows(ls):
            t, b = stage_tb(ls)
            return b * seq_len + s_base + t * C

        def issue_out(ls, p):
            pltpu.async_copy(gbufs[p][0], out_hbm.at[pl.ds(out_rows(ls), C)],
                             sem_o[p])

        def wait_out(ls, p):
            pltpu.make_async_copy(gbufs[p][0],
                                  out_hbm.at[pl.ds(out_rows(ls), C)],
                                  sem_o[p]).wait()

        def issue_pe(t):
            pltpu.async_copy(pe_hbm.at[pl.ds(s_base + t * C, C)], pe_v, sem_pe)

        def wait_pe(t):
            pltpu.make_async_copy(pe_hbm.at[pl.ds(s_base + t * C, C)], pe_v,
                                  sem_pe).wait()

        def compute(p):
            b1, b2, b3 = gbufs[p]

            def row(i, carry2):
                def grp(jj, carry3):
                    for u in range(4):
                        sl = pl.ds((jj * 4 + u) * L, L)
                        b1[i, sl] = (pe_v[i, sl] * w0 + b1[i, sl] * w1
                                     + b2[i, sl] * w2 + b3[i, sl] * w3)
                    return carry3
                return lax.fori_loop(0, groups // 4, grp, carry2)

            lax.fori_loop(0, C, row, 0)

        issue_pe(0)
        issue_gathers(0, 0)

        def iteration(k, carry):
            ls0 = 2 * k

            @pl.when(k > 0)
            def _():
                wait_out(ls0 - 1, 1)
            issue_gathers(ls0 + 1, 1)

            @pl.when(lax.rem(ls0, batch) == 0)
            def _():
                wait_pe(ls0 // batch)
            wait_gathers(ls0, 0)
            compute(0)

            @pl.when((lax.rem(ls0, batch) == batch - 1)
                     & (ls0 // batch + 1 < n_sc))
            def _():
                issue_pe(ls0 // batch + 1)
            issue_out(ls0, 0)

            ls1 = ls0 + 1

            wait_out(ls1 - 1, 0)
            @pl.when(k < n_stage // 2 - 1)
            def _():
                issue_gathers(ls1 + 1, 0)

            @pl.when(lax.rem(ls1, batch) == 0)
            def _():
                wait_pe(ls1 // batch)
            wait_gathers(ls1, 1)
            compute(1)

            @pl.when((lax.rem(ls1, batch) == batch - 1)
                     & (ls1 // batch + 1 < n_sc))
            def _():
                issue_pe(ls1 // batch + 1)
            issue_out(ls1, 1)
            return carry

        lax.fori_loop(0, n_stage // 2, iteration, 0)
        wait_out(n_stage - 1, 1)

    return body(idx_s, idx_r, idx_d, pe, st, rt, dt, wvec)


def kernel(positions, scales, rotations, distances, pe, scale_table,
           rotation_table, distance_table, fusion_weights):
    b, s = positions.shape
    d = pe.shape[1]
    NW = NC * NS
    n_sc = s // NW // C
    w = jax.nn.softmax(fusion_weights.astype(jnp.float32), axis=0)
    wvec = jnp.broadcast_to(w[:, None], (4, L)).astype(jnp.float32)
    shape = (b, NW, n_sc, C)
    idx_s = scales.reshape(shape).astype(jnp.int32).transpose(1, 0, 2, 3)
    idx_r = rotations.reshape(shape).astype(jnp.int32).transpose(1, 0, 2, 3)
    idx_d = distances.reshape(shape).astype(jnp.int32).transpose(1, 0, 2, 3)
    out = _sc_fused_lookup(idx_s, idx_r, idx_d, pe, scale_table,
                           rotation_table, distance_table, wvec,
                           batch=b, seq_len=s, d=d)
    return out.reshape(b, s, d)
